# baseline (speedup 1.0000x reference)
import functools

import jax
import jax.numpy as jnp
from jax import lax
from jax.experimental import pallas as pl
from jax.experimental.pallas import tpu as pltpu
from jax.experimental.pallas import tpu_sc as plsc

N = 10000
NR = 5
NA = 8
RA = NR * NA
T = 128
NROT = 8
NCLS = 10

NTILES = 32
NCH = 4
CW = 2560
NT = 9984
CWT = NT - 3 * CW
ITEMS_PER_TILE = RA * NCH // NTILES
NP = 10240


def _sc_interp_body(sig_hbm, bc_hbm, out_hbm, sig_v, bc_v, out_v, sem):
    cid = lax.axis_index("c")
    sid = lax.axis_index("s")
    wid = sid * 2 + cid
    pltpu.sync_copy(sig_hbm, sig_v)

    def item_body(j, carry):
        item = wid * ITEMS_PER_TILE + j
        ra = item // NCH
        ch = item - ra * NCH
        n0 = jnp.minimum(ch * CW, N - CW)
        base = ra * (6 * N) + n0
        hs = [pltpu.make_async_copy(
                  bc_hbm.at[pl.ds(base + q * N, CW)],
                  bc_v.at[pl.ds(q * CW, CW)], sem)
              for q in range(6)]
        for h in hs:
            h.start()
        for h in hs:
            h.wait()

        def grp(g2, c2):
            for u in range(4):
                g = g2 * 4 + u
                acc = jnp.zeros((16,), jnp.float32)
                for k in range(3):
                    iv = bc_v[pl.ds(2 * k * CW + g * 16, 16)]
                    wv = bc_v[pl.ds((2 * k + 1) * CW + g * 16, 16)]
                    sv = plsc.load_gather(sig_v, [iv.astype(jnp.int32)])
                    acc = acc + sv * wv
                out_v[pl.ds(g * 16, 16)] = acc
            return c2

        lax.fori_loop(0, CW // 64, grp, 0)

        pltpu.sync_copy(out_v, out_hbm.at[pl.ds(ra * NP + n0, CW)])
        return carry

    lax.fori_loop(0, ITEMS_PER_TILE, item_body, 0)


@jax.jit
def _sc_interp(sig, bc2f):
    mesh = plsc.VectorSubcoreMesh(core_axis_name="c", subcore_axis_name="s")
    fn = functools.partial(
        pl.kernel,
        out_type=jax.ShapeDtypeStruct((RA * NP,), jnp.float32),
        mesh=mesh,
        scratch_types=[
            pltpu.VMEM((N,), jnp.float32),
            pltpu.VMEM((6 * CW,), jnp.float32),
            pltpu.VMEM((CW,), jnp.float32),
            pltpu.SemaphoreType.DMA,
        ],
        compiler_params=pltpu.CompilerParams(needs_layout_passes=False),
    )(_sc_interp_body)
    return fn(sig, bc2f)


BN = 1250
NBLK = N // BN
BV = 2048


def _tc_conv_body(interp_ref, w2_ref, bt_ref, out_ref):
    jt = interp_ref[...]
    w2 = w2_ref[...]
    btc = bt_ref[...]

    best_ss = None
    best_act = None
    for o in range(NROT):
        if o == 0:
            wr = w2
        else:
            wr = jnp.concatenate([w2[:, o:, :], w2[:, :o, :]], axis=1)
        wrt = jnp.transpose(wr.reshape(RA, T), (1, 0))
        conv = lax.dot_general(wrt, jt, (((1,), (0,)), ((), ())),
                               preferred_element_type=jnp.float32) + btc
        act = jnp.maximum(conv, 0.0)
        ss = jnp.sum(act * act, axis=0, keepdims=True)
        if o == 0:
            best_ss, best_act = ss, act
        else:
            m = ss > best_ss
            best_act = jnp.where(m, act, best_act)
            best_ss = jnp.where(m, ss, best_ss)
    out_ref[...] = jnp.transpose(best_act, (1, 0))


@jax.jit
def _tc_conv(interp_t, w2, bt):
    return pl.pallas_call(
        _tc_conv_body,
        grid=(NP // BV,),
        in_specs=[
            pl.BlockSpec((RA, BV), lambda i: (0, i)),
            pl.BlockSpec((NR, NA, T), lambda i: (0, 0, 0)),
            pl.BlockSpec((T, 1), lambda i: (0, 0)),
        ],
        out_specs=pl.BlockSpec((BV, T), lambda i: (i, 0)),
        out_shape=jax.ShapeDtypeStruct((NP, T), jnp.float32),
    )(interp_t, w2, bt)


KB = BN * T


def _tc_dense_body(q_ref, wdt_ref, bd_ref, out_ref):
    i = pl.program_id(0)
    q = q_ref[...].reshape(1, KB)
    part = jnp.sum(wdt_ref[...] * q, axis=1, keepdims=True)

    @pl.when(i == 0)
    def _():
        out_ref[...] = bd_ref[...]

    out_ref[...] += part


@jax.jit
def _tc_dense(q25, wdt, bd):
    return pl.pallas_call(
        _tc_dense_body,
        grid=(NBLK,),
        in_specs=[
            pl.BlockSpec((1, 1, KB), lambda i: (i, 0, 0)),
            pl.BlockSpec((NCLS, KB), lambda i: (0, i)),
            pl.BlockSpec((NCLS, 1), lambda i: (0, 0)),
        ],
        out_specs=pl.BlockSpec((NCLS, 1), lambda i: (0, 0)),
        out_shape=jax.ShapeDtypeStruct((NCLS, 1), jnp.float32),
    )(q25, wdt, bd)


def kernel(signal, bc, W_templates, b_templates, W_dense, b_dense):
    sig = signal[0, :, 0]
    bc2f = jnp.transpose(bc, (0, 2, 3, 4, 5, 1)).reshape(RA * 6 * N)
    interp_t = _sc_interp(sig, bc2f).reshape(RA, NP)
    w2 = jnp.transpose(W_templates[:, :, :, 0], (1, 2, 0))
    pooled = _tc_conv(interp_t, w2, b_templates.reshape(T, 1))
    q25 = pooled[:N].reshape(NBLK, 1, KB)
    wdt = W_dense.T
    out = _tc_dense(q25, wdt, b_dense.reshape(NCLS, 1))
    return out.T


# speedup vs baseline: 1.0956x; 1.0956x over previous
import functools

import jax
import jax.numpy as jnp
from jax import lax
from jax.experimental import pallas as pl
from jax.experimental.pallas import tpu as pltpu
from jax.experimental.pallas import tpu_sc as plsc

N = 10000
NR = 5
NA = 8
RA = NR * NA
T = 128
NROT = 8
NCLS = 10

NTILES = 32
NCH = 4
CW = 2560
NT = 9984
CWT = NT - 3 * CW
ITEMS_PER_TILE = RA * NCH // NTILES
NP = 10240


def _sc_interp_body(sig_hbm, bc_hbm, out_hbm, sig_v,
                    bc_a, bc_b, out_a, out_b, sem_a, sem_b, osem_a, osem_b):
    cid = lax.axis_index("c")
    sid = lax.axis_index("s")
    wid = sid * 2 + cid
    bufs = (bc_a, bc_b)
    obufs = (out_a, out_b)
    sems = (sem_a, sem_b)
    osems = (osem_a, osem_b)

    def item_addr(j):
        item = wid * ITEMS_PER_TILE + j
        ra = item // NCH
        ch = item - ra * NCH
        n0 = jnp.minimum(ch * CW, N - CW)
        return ra, n0

    def fire(j, ra, n0):
        base = ra * (6 * N) + n0
        for q in range(6):
            pltpu.make_async_copy(
                bc_hbm.at[pl.ds(base + q * N, CW)],
                bufs[j % 2].at[pl.ds(q * CW, CW)], sems[j % 2]).start()

    addrs = [item_addr(j) for j in range(ITEMS_PER_TILE)]
    fire(0, *addrs[0])
    pltpu.sync_copy(sig_hbm, sig_v)

    for j in range(ITEMS_PER_TILE):
        if j + 1 < ITEMS_PER_TILE:
            fire(j + 1, *addrs[j + 1])
        bc_v = bufs[j % 2]
        out_v = obufs[j % 2]
        pltpu.make_async_copy(bc_hbm.at[pl.ds(0, 6 * CW)], bc_v,
                              sems[j % 2]).wait()
        if j >= 2:
            pltpu.make_async_copy(out_v, out_hbm.at[pl.ds(0, CW)],
                                  osems[j % 2]).wait()

        def grp(g2, c2, bc_v=bc_v, out_v=out_v):
            for u in range(4):
                g = g2 * 4 + u
                acc = jnp.zeros((16,), jnp.float32)
                for k in range(3):
                    iv = bc_v[pl.ds(2 * k * CW + g * 16, 16)]
                    wv = bc_v[pl.ds((2 * k + 1) * CW + g * 16, 16)]
                    sv = plsc.load_gather(sig_v, [iv.astype(jnp.int32)])
                    acc = acc + sv * wv
                out_v[pl.ds(g * 16, 16)] = acc
            return c2

        lax.fori_loop(0, CW // 64, grp, 0)
        ra, n0 = addrs[j]
        pltpu.make_async_copy(out_v, out_hbm.at[pl.ds(ra * NP + n0, CW)],
                              osems[j % 2]).start()

    pltpu.make_async_copy(out_a, out_hbm.at[pl.ds(0, CW)], osem_a).wait()
    pltpu.make_async_copy(out_b, out_hbm.at[pl.ds(0, CW)], osem_b).wait()


@jax.jit
def _sc_interp(sig, bc2f):
    mesh = plsc.VectorSubcoreMesh(core_axis_name="c", subcore_axis_name="s")
    fn = functools.partial(
        pl.kernel,
        out_type=jax.ShapeDtypeStruct((RA * NP,), jnp.float32),
        mesh=mesh,
        scratch_types=[
            pltpu.VMEM((N,), jnp.float32),
            pltpu.VMEM((6 * CW,), jnp.float32),
            pltpu.VMEM((6 * CW,), jnp.float32),
            pltpu.VMEM((CW,), jnp.float32),
            pltpu.VMEM((CW,), jnp.float32),
            pltpu.SemaphoreType.DMA,
            pltpu.SemaphoreType.DMA,
            pltpu.SemaphoreType.DMA,
            pltpu.SemaphoreType.DMA,
        ],
        compiler_params=pltpu.CompilerParams(needs_layout_passes=False),
    )(_sc_interp_body)
    return fn(sig, bc2f)


BN = 1250
NBLK = N // BN
BV = 2048


def _tc_conv_body(interp_ref, w2_ref, bt_ref, out_ref):
    jt = interp_ref[...]
    w2 = w2_ref[...]
    btc = bt_ref[...]

    best_ss = None
    best_act = None
    for o in range(NROT):
        if o == 0:
            wr = w2
        else:
            wr = jnp.concatenate([w2[:, o:, :], w2[:, :o, :]], axis=1)
        wrt = jnp.transpose(wr.reshape(RA, T), (1, 0))
        conv = lax.dot_general(wrt, jt, (((1,), (0,)), ((), ())),
                               preferred_element_type=jnp.float32) + btc
        act = jnp.maximum(conv, 0.0)
        ss = jnp.sum(act * act, axis=0, keepdims=True)
        if o == 0:
            best_ss, best_act = ss, act
        else:
            m = ss > best_ss
            best_act = jnp.where(m, act, best_act)
            best_ss = jnp.where(m, ss, best_ss)
    out_ref[...] = jnp.transpose(best_act, (1, 0))


@jax.jit
def _tc_conv(interp_t, w2, bt):
    return pl.pallas_call(
        _tc_conv_body,
        grid=(NP // BV,),
        in_specs=[
            pl.BlockSpec((RA, BV), lambda i: (0, i)),
            pl.BlockSpec((NR, NA, T), lambda i: (0, 0, 0)),
            pl.BlockSpec((T, 1), lambda i: (0, 0)),
        ],
        out_specs=pl.BlockSpec((BV, T), lambda i: (i, 0)),
        out_shape=jax.ShapeDtypeStruct((N, T), jnp.float32),
    )(interp_t, w2, bt)


KB = BN * T


def _tc_dense_body(q_ref, wdt_ref, bd_ref, out_ref):
    i = pl.program_id(0)
    q = q_ref[...].reshape(1, KB)
    part = jnp.sum(wdt_ref[...] * q, axis=1, keepdims=True)

    @pl.when(i == 0)
    def _():
        out_ref[...] = bd_ref[...]

    out_ref[...] += part


@jax.jit
def _tc_dense(q25, wdt, bd):
    return pl.pallas_call(
        _tc_dense_body,
        grid=(NBLK,),
        in_specs=[
            pl.BlockSpec((1, 1, KB), lambda i: (i, 0, 0)),
            pl.BlockSpec((NCLS, KB), lambda i: (0, i)),
            pl.BlockSpec((NCLS, 1), lambda i: (0, 0)),
        ],
        out_specs=pl.BlockSpec((NCLS, 1), lambda i: (0, 0)),
        out_shape=jax.ShapeDtypeStruct((NCLS, 1), jnp.float32),
    )(q25, wdt, bd)


def kernel(signal, bc, W_templates, b_templates, W_dense, b_dense):
    sig = signal[0, :, 0]
    bc2f = jnp.transpose(bc, (0, 2, 3, 4, 5, 1)).reshape(RA * 6 * N)
    interp_t = _sc_interp(sig, bc2f).reshape(RA, NP)
    w2 = jnp.transpose(W_templates[:, :, :, 0], (1, 2, 0))
    pooled = _tc_conv(interp_t, w2, b_templates.reshape(T, 1))
    q25 = pooled.reshape(NBLK, 1, KB)
    wdt = W_dense.T
    out = _tc_dense(q25, wdt, b_dense.reshape(NCLS, 1))
    return out.T


# speedup vs baseline: 1.2027x; 1.0978x over previous
import functools

import jax
import jax.numpy as jnp
from jax import lax
from jax.experimental import pallas as pl
from jax.experimental.pallas import tpu as pltpu
from jax.experimental.pallas import tpu_sc as plsc

N = 10000
NR = 5
NA = 8
RA = NR * NA
T = 128
NROT = 8
NCLS = 10

NTILES = 32
NCH = 4
CW = 1280
ITEMS_PER_TILE = RA * NCH // NTILES
VH = 5120
H2 = N - VH


def _sc_interp_body(nmax, sig_hbm, bc_hbm, out_hbm, sig_v,
                    bc_a, bc_b, out_a, out_b, sem_a, sem_b, osem_a, osem_b):
    cid = lax.axis_index("c")
    sid = lax.axis_index("s")
    wid = sid * 2 + cid
    bufs = (bc_a, bc_b)
    obufs = (out_a, out_b)
    sems = (sem_a, sem_b)
    osems = (osem_a, osem_b)

    def item_addr(j):
        item = wid * ITEMS_PER_TILE + j
        ra = item // NCH
        ch = item - ra * NCH
        n0 = jnp.minimum(ch * CW, nmax - CW)
        return ra, n0

    def fire(j, ra, n0):
        base = ra * (6 * nmax) + n0
        for q in range(6):
            pltpu.make_async_copy(
                bc_hbm.at[pl.ds(base + q * nmax, CW)],
                bufs[j % 2].at[pl.ds(q * CW, CW)], sems[j % 2]).start()

    addrs = [item_addr(j) for j in range(ITEMS_PER_TILE)]
    fire(0, *addrs[0])
    pltpu.sync_copy(sig_hbm, sig_v)

    for j in range(ITEMS_PER_TILE):
        if j + 1 < ITEMS_PER_TILE:
            fire(j + 1, *addrs[j + 1])
        bc_v = bufs[j % 2]
        out_v = obufs[j % 2]
        pltpu.make_async_copy(bc_hbm.at[pl.ds(0, 6 * CW)], bc_v,
                              sems[j % 2]).wait()
        if j >= 2:
            pltpu.make_async_copy(out_v, out_hbm.at[pl.ds(0, CW)],
                                  osems[j % 2]).wait()

        def grp(g2, c2, bc_v=bc_v, out_v=out_v):
            for u in range(4):
                g = g2 * 4 + u
                acc = jnp.zeros((16,), jnp.float32)
                for k in range(3):
                    iv = bc_v[pl.ds(2 * k * CW + g * 16, 16)]
                    wv = bc_v[pl.ds((2 * k + 1) * CW + g * 16, 16)]
                    sv = plsc.load_gather(sig_v, [iv.astype(jnp.int32)])
                    acc = acc + sv * wv
                out_v[pl.ds(g * 16, 16)] = acc
            return c2

        lax.fori_loop(0, CW // 64, grp, 0)
        ra, n0 = addrs[j]
        pltpu.make_async_copy(out_v, out_hbm.at[pl.ds(ra * VH + n0, CW)],
                              osems[j % 2]).start()

    pltpu.make_async_copy(out_a, out_hbm.at[pl.ds(0, CW)], osem_a).wait()
    pltpu.make_async_copy(out_b, out_hbm.at[pl.ds(0, CW)], osem_b).wait()


def _sc_interp(sig, bc2f, nmax):
    mesh = plsc.VectorSubcoreMesh(core_axis_name="c", subcore_axis_name="s")
    fn = functools.partial(
        pl.kernel,
        out_type=jax.ShapeDtypeStruct((RA * VH,), jnp.float32),
        mesh=mesh,
        scratch_types=[
            pltpu.VMEM((N,), jnp.float32),
            pltpu.VMEM((6 * CW,), jnp.float32),
            pltpu.VMEM((6 * CW,), jnp.float32),
            pltpu.VMEM((CW,), jnp.float32),
            pltpu.VMEM((CW,), jnp.float32),
            pltpu.SemaphoreType.DMA,
            pltpu.SemaphoreType.DMA,
            pltpu.SemaphoreType.DMA,
            pltpu.SemaphoreType.DMA,
        ],
        compiler_params=pltpu.CompilerParams(needs_layout_passes=False),
    )(functools.partial(_sc_interp_body, nmax))
    return fn(sig, bc2f)


BN = 1250
NBLK = N // BN
BV = 1024


def _tc_conv_body(interp_ref, w2_ref, bt_ref, out_ref):
    jt = interp_ref[...]
    w2 = w2_ref[...]
    btc = bt_ref[...]

    best_ss = None
    best_act = None
    for o in range(NROT):
        if o == 0:
            wr = w2
        else:
            wr = jnp.concatenate([w2[:, o:, :], w2[:, :o, :]], axis=1)
        wrt = jnp.transpose(wr.reshape(RA, T), (1, 0))
        conv = lax.dot_general(wrt, jt, (((1,), (0,)), ((), ())),
                               preferred_element_type=jnp.float32) + btc
        act = jnp.maximum(conv, 0.0)
        ss = jnp.sum(act * act, axis=0, keepdims=True)
        if o == 0:
            best_ss, best_act = ss, act
        else:
            m = ss > best_ss
            best_act = jnp.where(m, act, best_act)
            best_ss = jnp.where(m, ss, best_ss)
    out_ref[...] = jnp.transpose(best_act, (1, 0))


def _tc_conv(interp_t, w2, bt, nout):
    return pl.pallas_call(
        _tc_conv_body,
        grid=(VH // BV,),
        in_specs=[
            pl.BlockSpec((RA, BV), lambda i: (0, i)),
            pl.BlockSpec((NR, NA, T), lambda i: (0, 0, 0)),
            pl.BlockSpec((T, 1), lambda i: (0, 0)),
        ],
        out_specs=pl.BlockSpec((BV, T), lambda i: (i, 0)),
        out_shape=jax.ShapeDtypeStruct((nout, T), jnp.float32),
    )(interp_t, w2, bt)


KB = BN * T


def _tc_dense_body(q_ref, wdt_ref, bd_ref, out_ref):
    i = pl.program_id(0)
    q = q_ref[...].reshape(1, KB)
    part = jnp.sum(wdt_ref[...] * q, axis=1, keepdims=True)

    @pl.when(i == 0)
    def _():
        out_ref[...] = bd_ref[...]

    out_ref[...] += part


@jax.jit
def _tc_dense(q25, wdt, bd):
    return pl.pallas_call(
        _tc_dense_body,
        grid=(NBLK,),
        in_specs=[
            pl.BlockSpec((1, 1, KB), lambda i: (i, 0, 0)),
            pl.BlockSpec((NCLS, KB), lambda i: (0, i)),
            pl.BlockSpec((NCLS, 1), lambda i: (0, 0)),
        ],
        out_specs=pl.BlockSpec((NCLS, 1), lambda i: (0, 0)),
        out_shape=jax.ShapeDtypeStruct((NCLS, 1), jnp.float32),
    )(q25, wdt, bd)


def kernel(signal, bc, W_templates, b_templates, W_dense, b_dense):
    sig = signal[0, :, 0]
    bc6 = jnp.transpose(bc, (0, 2, 3, 4, 5, 1))
    bc2f_a = bc6[..., :VH].reshape(RA * 6 * VH)
    bc2f_b = bc6[..., VH:].reshape(RA * 6 * H2)
    w2 = jnp.transpose(W_templates[:, :, :, 0], (1, 2, 0))
    btc = b_templates.reshape(T, 1)

    it_a = _sc_interp(sig, bc2f_a, VH).reshape(RA, VH)
    it_b = _sc_interp(sig, bc2f_b, H2).reshape(RA, VH)
    pooled_a = _tc_conv(it_a, w2, btc, VH)
    pooled_b = _tc_conv(it_b, w2, btc, H2)

    pooled = jnp.concatenate([pooled_a, pooled_b], axis=0)
    q25 = pooled.reshape(NBLK, 1, KB)
    wdt = W_dense.T
    out = _tc_dense(q25, wdt, b_dense.reshape(NCLS, 1))
    return out.T


# speedup vs baseline: 1.2137x; 1.0092x over previous
import functools

import jax
import jax.numpy as jnp
from jax import lax
from jax.experimental import pallas as pl
from jax.experimental.pallas import tpu as pltpu
from jax.experimental.pallas import tpu_sc as plsc

N = 10000
NR = 5
NA = 8
RA = NR * NA
T = 128
NROT = 8
NCLS = 10

NTILES = 32
NCH = 4
CW = 1280
ITEMS_PER_TILE = RA * NCH // NTILES
VH = 5120
H2 = N - VH


def _sc_interp_body(nmax, sig_hbm, bc_hbm, out_hbm, sig_v,
                    bc_a, bc_b, out_a, out_b, sem_a, sem_b, osem_a, osem_b):
    cid = lax.axis_index("c")
    sid = lax.axis_index("s")
    wid = sid * 2 + cid
    bufs = (bc_a, bc_b)
    obufs = (out_a, out_b)
    sems = (sem_a, sem_b)
    osems = (osem_a, osem_b)

    def item_addr(j):
        item = wid * ITEMS_PER_TILE + j
        ra = item // NCH
        ch = item - ra * NCH
        n0 = jnp.minimum(ch * CW, nmax - CW)
        return ra, n0

    def fire(j, ra, n0):
        base = ra * (6 * nmax) + n0
        for q in range(6):
            pltpu.make_async_copy(
                bc_hbm.at[pl.ds(base + q * nmax, CW)],
                bufs[j % 2].at[pl.ds(q * CW, CW)], sems[j % 2]).start()

    addrs = [item_addr(j) for j in range(ITEMS_PER_TILE)]
    fire(0, *addrs[0])
    pltpu.sync_copy(sig_hbm, sig_v)

    for j in range(ITEMS_PER_TILE):
        if j + 1 < ITEMS_PER_TILE:
            fire(j + 1, *addrs[j + 1])
        bc_v = bufs[j % 2]
        out_v = obufs[j % 2]
        pltpu.make_async_copy(bc_hbm.at[pl.ds(0, 6 * CW)], bc_v,
                              sems[j % 2]).wait()
        if j >= 2:
            pltpu.make_async_copy(out_v, out_hbm.at[pl.ds(0, CW)],
                                  osems[j % 2]).wait()

        def grp(g2, c2, bc_v=bc_v, out_v=out_v):
            for u in range(4):
                g = g2 * 4 + u
                acc = jnp.zeros((16,), jnp.float32)
                for k in range(3):
                    iv = bc_v[pl.ds(2 * k * CW + g * 16, 16)]
                    wv = bc_v[pl.ds((2 * k + 1) * CW + g * 16, 16)]
                    sv = plsc.load_gather(sig_v, [iv.astype(jnp.int32)])
                    acc = acc + sv * wv
                out_v[pl.ds(g * 16, 16)] = acc
            return c2

        lax.fori_loop(0, CW // 64, grp, 0)
        ra, n0 = addrs[j]
        pltpu.make_async_copy(out_v, out_hbm.at[pl.ds(ra * VH + n0, CW)],
                              osems[j % 2]).start()

    pltpu.make_async_copy(out_a, out_hbm.at[pl.ds(0, CW)], osem_a).wait()
    pltpu.make_async_copy(out_b, out_hbm.at[pl.ds(0, CW)], osem_b).wait()


def _sc_interp(sig, bc2f, nmax):
    mesh = plsc.VectorSubcoreMesh(core_axis_name="c", subcore_axis_name="s")
    fn = functools.partial(
        pl.kernel,
        out_type=jax.ShapeDtypeStruct((RA * VH,), jnp.float32),
        mesh=mesh,
        scratch_types=[
            pltpu.VMEM((N,), jnp.float32),
            pltpu.VMEM((6 * CW,), jnp.float32),
            pltpu.VMEM((6 * CW,), jnp.float32),
            pltpu.VMEM((CW,), jnp.float32),
            pltpu.VMEM((CW,), jnp.float32),
            pltpu.SemaphoreType.DMA,
            pltpu.SemaphoreType.DMA,
            pltpu.SemaphoreType.DMA,
            pltpu.SemaphoreType.DMA,
        ],
        compiler_params=pltpu.CompilerParams(needs_layout_passes=False),
    )(functools.partial(_sc_interp_body, nmax))
    return fn(sig, bc2f)


BN = 1250
NBLK = N // BN
BV = 2560


def _tc_conv_body(interp_ref, w2_ref, bt_ref, out_ref):
    jt = interp_ref[...]
    w2 = w2_ref[...]
    btc = bt_ref[...]

    best_ss = None
    best_act = None
    for o in range(NROT):
        if o == 0:
            wr = w2
        else:
            wr = jnp.concatenate([w2[:, o:, :], w2[:, :o, :]], axis=1)
        wrt = jnp.transpose(wr.reshape(RA, T), (1, 0))
        conv = lax.dot_general(wrt, jt, (((1,), (0,)), ((), ())),
                               preferred_element_type=jnp.float32) + btc
        act = jnp.maximum(conv, 0.0)
        ss = jnp.sum(act * act, axis=0, keepdims=True)
        if o == 0:
            best_ss, best_act = ss, act
        else:
            m = ss > best_ss
            best_act = jnp.where(m, act, best_act)
            best_ss = jnp.where(m, ss, best_ss)
    out_ref[...] = jnp.transpose(best_act, (1, 0))


def _tc_conv(interp_t, w2, bt, nout):
    return pl.pallas_call(
        _tc_conv_body,
        grid=(VH // BV,),
        in_specs=[
            pl.BlockSpec((RA, BV), lambda i: (0, i)),
            pl.BlockSpec((NR, NA, T), lambda i: (0, 0, 0)),
            pl.BlockSpec((T, 1), lambda i: (0, 0)),
        ],
        out_specs=pl.BlockSpec((BV, T), lambda i: (i, 0)),
        out_shape=jax.ShapeDtypeStruct((nout, T), jnp.float32),
    )(interp_t, w2, bt)


KB = BN * T


def _tc_dense_body(q_ref, wdt_ref, bd_ref, out_ref):
    i = pl.program_id(0)
    q = q_ref[...].reshape(1, KB)
    part = jnp.sum(wdt_ref[...] * q, axis=1, keepdims=True)

    @pl.when(i == 0)
    def _():
        out_ref[...] = bd_ref[...]

    out_ref[...] += part


@jax.jit
def _tc_dense(q25, wdt, bd):
    return pl.pallas_call(
        _tc_dense_body,
        grid=(NBLK,),
        in_specs=[
            pl.BlockSpec((1, 1, KB), lambda i: (i, 0, 0)),
            pl.BlockSpec((NCLS, KB), lambda i: (0, i)),
            pl.BlockSpec((NCLS, 1), lambda i: (0, 0)),
        ],
        out_specs=pl.BlockSpec((NCLS, 1), lambda i: (0, 0)),
        out_shape=jax.ShapeDtypeStruct((NCLS, 1), jnp.float32),
    )(q25, wdt, bd)


def kernel(signal, bc, W_templates, b_templates, W_dense, b_dense):
    sig = signal[0, :, 0]
    bc6 = jnp.transpose(bc, (0, 2, 3, 4, 5, 1))
    bc2f_a = bc6[..., :VH].reshape(RA * 6 * VH)
    bc2f_b = bc6[..., VH:].reshape(RA * 6 * H2)
    w2 = jnp.transpose(W_templates[:, :, :, 0], (1, 2, 0))
    btc = b_templates.reshape(T, 1)

    it_a = _sc_interp(sig, bc2f_a, VH).reshape(RA, VH)
    it_b = _sc_interp(sig, bc2f_b, H2).reshape(RA, VH)
    pooled_a = _tc_conv(it_a, w2, btc, VH)
    pooled_b = _tc_conv(it_b, w2, btc, H2)

    pooled = jnp.concatenate([pooled_a, pooled_b], axis=0)
    q25 = pooled.reshape(NBLK, 1, KB)
    wdt = W_dense.T
    out = _tc_dense(q25, wdt, b_dense.reshape(NCLS, 1))
    return out.T
